# Initial kernel scaffold; baseline (speedup 1.0000x reference)
import jax
import jax.numpy as jnp
from jax.experimental import pallas as pl


def kernel(S, positions, edge_index):
    raise NotImplementedError("write your pallas kernel here")


# speedup vs baseline: 83.9430x; 83.9430x over previous
import functools

import jax
import jax.numpy as jnp
from jax import lax
from jax.experimental import pallas as pl
from jax.experimental.pallas import tpu as pltpu
from jax.experimental.pallas import tpu_sc as plsc

_WEIGHT = 0.01

_SUB = 128
_SUPER = 32
_HALF = 4
_WIDTH = 32


@functools.partial(jax.jit, static_argnums=(3, 4))
def _edge_loss_sums(table, src_idx, dst_idx, n_rows, n_workers):
    mesh = plsc.VectorSubcoreMesh(
        core_axis_name="c", subcore_axis_name="s", num_cores=2, num_subcores=16
    )
    oct_total = n_rows // 8
    base_oct = oct_total // n_workers
    rem_oct = oct_total - base_oct * n_workers
    max_cnt = (base_oct + (1 if rem_oct else 0)) * 8
    n_super = (max_cnt + _SUPER - 1) // _SUPER
    n_halves = _SUPER // _HALF

    @functools.partial(
        pl.kernel,
        out_type=jax.ShapeDtypeStruct((n_workers * 16,), jnp.float32),
        mesh=mesh,
        scratch_types=[
            pltpu.VMEM((_SUPER * _SUB,), jnp.int32),
            pltpu.VMEM((_SUPER * _SUB,), jnp.int32),
            pltpu.VMEM((2, _HALF * _SUB, _WIDTH), jnp.float32),
            pltpu.VMEM((2, _HALF * _SUB, _WIDTH), jnp.float32),
            pltpu.VMEM((16,), jnp.float32),
            pltpu.SemaphoreType.DMA,
            pltpu.SemaphoreType.DMA,
        ],
        compiler_params=pltpu.CompilerParams(use_tc_tiling_on_sc=False),
    )
    def k(table_h, src_h, dst_h, out_h, idx_s, idx_d, rows_s, rows_d, res_v,
          sem0, sem1):
        wid = lax.axis_index("s") * 2 + lax.axis_index("c")
        lo = (wid * base_oct + jnp.minimum(wid, rem_oct)) * 8
        hi = lo + (base_oct + jnp.where(wid < rem_oct, 1, 0)) * 8
        sems = (sem0, sem1)

        def super_body(sg, acc):
            g = lo + sg * _SUPER
            n_full = hi - g

            @pl.when(n_full >= _SUPER)
            def _():
                pltpu.sync_copy(src_h.at[pl.ds(g * _SUB, _SUPER * _SUB)],
                                idx_s)
                pltpu.sync_copy(dst_h.at[pl.ds(g * _SUB, _SUPER * _SUB)],
                                idx_d)

            @pl.when(n_full < _SUPER)
            def _():
                for r8 in range(0, _SUPER, 8):
                    @pl.when(r8 < n_full)
                    def _(r8=r8):
                        pltpu.sync_copy(
                            src_h.at[pl.ds((g + r8) * _SUB, 8 * _SUB)],
                            idx_s.at[pl.ds(r8 * _SUB, 8 * _SUB)])
                        pltpu.sync_copy(
                            dst_h.at[pl.ds((g + r8) * _SUB, 8 * _SUB)],
                            idx_d.at[pl.ds(r8 * _SUB, 8 * _SUB)])

            def fire(h):
                b = h % 2
                descs = []
                for j in range(_HALF):
                    r = h * _HALF + j
                    cond = g + r < hi
                    d1 = pltpu.make_async_copy(
                        table_h.at[idx_s.at[pl.ds(r * _SUB, _SUB)]],
                        rows_s.at[b, pl.ds(j * _SUB, _SUB)], sems[b])
                    d2 = pltpu.make_async_copy(
                        table_h.at[idx_d.at[pl.ds(r * _SUB, _SUB)]],
                        rows_d.at[b, pl.ds(j * _SUB, _SUB)], sems[b])

                    @pl.when(cond)
                    def _(d1=d1, d2=d2):
                        d1.start()
                        d2.start()

                    descs.append((cond, d1, d2))
                return descs

            def drain(descs):
                for cond, d1, d2 in descs:
                    @pl.when(cond)
                    def _(d1=d1, d2=d2):
                        d1.wait()
                        d2.wait()

            def compute(h, acc):
                b = h % 2
                n_e = jnp.clip(hi - (g + h * _HALF), 0, _HALF) * _SUB
                rs = rows_s.at[b]
                rd = rows_d.at[b]

                def edge_body(e, a):
                    sa = rs[e, pl.ds(0, 16)]
                    sb = rd[e, pl.ds(0, 16)]
                    d = rs[e, pl.ds(16, 16)] - rd[e, pl.ds(16, 16)]
                    sq = d * d
                    dist2 = sq[0] + sq[1] + sq[2]
                    prod = jnp.maximum(sa, 0.0) * jnp.maximum(sb, 0.0)
                    return a + prod * dist2

                return lax.fori_loop(0, n_e, edge_body, acc)

            descs = fire(0)
            for h in range(n_halves):
                nxt = fire(h + 1) if h + 1 < n_halves else []
                drain(descs)
                acc = compute(h, acc)
                descs = nxt
            return acc

        acc = lax.fori_loop(0, n_super, super_body,
                            jnp.zeros((16,), jnp.float32))
        res_v[...] = acc
        pltpu.sync_copy(res_v, out_h.at[pl.ds(wid * 16, 16)])

    return k(table, src_idx, dst_idx)


def kernel(S, positions, edge_index):
    n, k = S.shape
    num_edges = edge_index.shape[1]
    table = jnp.concatenate(
        [S, positions.astype(jnp.float32),
         jnp.zeros((n, _WIDTH - k - 3), jnp.float32)], axis=1)
    ei = edge_index.astype(jnp.int32)
    partial = _edge_loss_sums(table, ei[0], ei[1], num_edges // _SUB, 32)
    return _WEIGHT * jnp.sum(partial) / num_edges


# speedup vs baseline: 102.1219x; 1.2166x over previous
import functools

import jax
import jax.numpy as jnp
from jax import lax
from jax.experimental import pallas as pl
from jax.experimental.pallas import tpu as pltpu
from jax.experimental.pallas import tpu_sc as plsc

_WEIGHT = 0.01

_SUB = 128
_SUPER = 32
_HALF = 4
_WIDTH = 32
_UNROLL = 8


@functools.partial(jax.jit, static_argnums=(2, 3))
def _edge_loss_sums(table, edge_idx, n_rows, n_workers):
    mesh = plsc.VectorSubcoreMesh(
        core_axis_name="c", subcore_axis_name="s", num_cores=2, num_subcores=16
    )
    oct_total = n_rows // 8
    base_oct = oct_total // n_workers
    rem_oct = oct_total - base_oct * n_workers
    max_cnt = (base_oct + (1 if rem_oct else 0)) * 8
    n_super = (max_cnt + _SUPER - 1) // _SUPER
    n_halves = _SUPER // _HALF

    @functools.partial(
        pl.kernel,
        out_type=jax.ShapeDtypeStruct((n_workers * 16,), jnp.float32),
        mesh=mesh,
        scratch_types=[
            pltpu.VMEM((_SUPER * _SUB,), jnp.int32),
            pltpu.VMEM((_SUPER * _SUB,), jnp.int32),
            pltpu.VMEM((2, _HALF * _SUB, _WIDTH), jnp.float32),
            pltpu.VMEM((2, _HALF * _SUB, _WIDTH), jnp.float32),
            pltpu.VMEM((16,), jnp.float32),
            pltpu.SemaphoreType.DMA,
            pltpu.SemaphoreType.DMA,
        ],
        compiler_params=pltpu.CompilerParams(use_tc_tiling_on_sc=False),
    )
    def k(table_h, edge_h, out_h, idx_s, idx_d, rows_s, rows_d, res_v,
          sem0, sem1):
        src_h = edge_h.at[0]
        dst_h = edge_h.at[1]
        wid = lax.axis_index("s") * 2 + lax.axis_index("c")
        lo = (wid * base_oct + jnp.minimum(wid, rem_oct)) * 8
        hi = lo + (base_oct + jnp.where(wid < rem_oct, 1, 0)) * 8
        sems = (sem0, sem1)

        def super_body(sg, acc):
            g = lo + sg * _SUPER
            n_full = hi - g

            @pl.when(n_full >= _SUPER)
            def _():
                pltpu.sync_copy(src_h.at[pl.ds(g * _SUB, _SUPER * _SUB)],
                                idx_s)
                pltpu.sync_copy(dst_h.at[pl.ds(g * _SUB, _SUPER * _SUB)],
                                idx_d)

            @pl.when(n_full < _SUPER)
            def _():
                for r8 in range(0, _SUPER, 8):
                    @pl.when(r8 < n_full)
                    def _(r8=r8):
                        pltpu.sync_copy(
                            src_h.at[pl.ds((g + r8) * _SUB, 8 * _SUB)],
                            idx_s.at[pl.ds(r8 * _SUB, 8 * _SUB)])
                        pltpu.sync_copy(
                            dst_h.at[pl.ds((g + r8) * _SUB, 8 * _SUB)],
                            idx_d.at[pl.ds(r8 * _SUB, 8 * _SUB)])

            def fire(h):
                b = h % 2
                descs = []
                for j in range(_HALF):
                    r = h * _HALF + j
                    cond = g + r < hi
                    d1 = pltpu.make_async_copy(
                        table_h.at[idx_s.at[pl.ds(r * _SUB, _SUB)]],
                        rows_s.at[b, pl.ds(j * _SUB, _SUB)], sems[b])
                    d2 = pltpu.make_async_copy(
                        table_h.at[idx_d.at[pl.ds(r * _SUB, _SUB)]],
                        rows_d.at[b, pl.ds(j * _SUB, _SUB)], sems[b])

                    @pl.when(cond)
                    def _(d1=d1, d2=d2):
                        d1.start()
                        d2.start()

                    descs.append((cond, d1, d2))
                return descs

            def drain(descs):
                for cond, d1, d2 in descs:
                    @pl.when(cond)
                    def _(d1=d1, d2=d2):
                        d1.wait()
                        d2.wait()

            def compute(h, acc):
                b = h % 2
                n_e = jnp.clip(hi - (g + h * _HALF), 0, _HALF) * _SUB
                rs = rows_s.at[b]
                rd = rows_d.at[b]

                def edge_group_body(i, a):
                    e0 = i * _UNROLL
                    for u in range(_UNROLL):
                        e = e0 + u
                        sa = rs[e, pl.ds(0, 16)]
                        sb = rd[e, pl.ds(0, 16)]
                        d = rs[e, pl.ds(16, 16)] - rd[e, pl.ds(16, 16)]
                        sq = d * d
                        dist2 = sq[0] + sq[1] + sq[2]
                        prod = jnp.maximum(sa, 0.0) * jnp.maximum(sb, 0.0)
                        a = a + prod * dist2
                    return a

                return lax.fori_loop(0, n_e // _UNROLL, edge_group_body, acc)

            descs = fire(0)
            for h in range(n_halves):
                nxt = fire(h + 1) if h + 1 < n_halves else []
                drain(descs)
                acc = compute(h, acc)
                descs = nxt
            return acc

        acc = lax.fori_loop(0, n_super, super_body,
                            jnp.zeros((16,), jnp.float32))
        res_v[...] = acc
        pltpu.sync_copy(res_v, out_h.at[pl.ds(wid * 16, 16)])

    return k(table, edge_idx)


def kernel(S, positions, edge_index):
    n, k = S.shape
    num_edges = edge_index.shape[1]
    table = jnp.concatenate(
        [S, positions.astype(jnp.float32),
         jnp.zeros((n, _WIDTH - k - 3), jnp.float32)], axis=1)
    ei = edge_index.astype(jnp.int32)
    partial = _edge_loss_sums(table, ei, num_edges // _SUB, 32)
    return _WEIGHT * jnp.sum(partial) / num_edges
